# Initial kernel scaffold; baseline (speedup 1.0000x reference)
import jax
import jax.numpy as jnp
from jax.experimental import pallas as pl


def kernel(inputs, lengths, emb, Wih0, Whh0, bih0, bhh0, Wih1, Whh1, bih1, bhh1):
    raise NotImplementedError("write your pallas kernel here")


# speedup vs baseline: 2.7605x; 2.7605x over previous
import functools

import jax
import jax.numpy as jnp
from jax import lax
from jax.experimental import pallas as pl
from jax.experimental.pallas import tpu as pltpu
from jax.experimental.pallas import tpu_sc as plsc

_EMB = 128
_HID = 512
_CHUNK = 104
_NCHUNK = 2
_ROWS_W = _CHUNK * _NCHUNK


def _emb_gather(table, idx3):
    nw = idx3.shape[0]
    mesh = plsc.VectorSubcoreMesh(core_axis_name="c", subcore_axis_name="s")

    @functools.partial(
        pl.kernel,
        mesh=mesh,
        out_type=jax.ShapeDtypeStruct((nw * _ROWS_W, _EMB), jnp.float32),
        scratch_types=[
            pltpu.VMEM((_NCHUNK, _CHUNK), jnp.int32),
            pltpu.VMEM((_ROWS_W, _EMB), jnp.float32),
            pltpu.SemaphoreType.DMA,
        ],
    )
    def gather_kernel(table_hbm, idx_hbm, out_hbm, idx_v, rows_v, sem):
        nc = lax.axis_size("c")
        wid = lax.axis_index("s") * nc + lax.axis_index("c")
        pltpu.sync_copy(idx_hbm.at[wid], idx_v)
        cps = []
        for j in range(_NCHUNK):
            cps.append(
                pltpu.async_copy(
                    table_hbm.at[idx_v.at[j]],
                    rows_v.at[pl.ds(j * _CHUNK, _CHUNK)],
                    sem,
                )
            )
        for cp in cps:
            cp.wait()
        pltpu.sync_copy(rows_v, out_hbm.at[pl.ds(wid * _ROWS_W, _ROWS_W)])

    return gather_kernel(table, idx3)


def _lstm2_body(net_ref, len_ref, eos_ref, w0_ref, w1_ref, b0_ref, b1_ref,
                ys_ref, h0_ref, c0_ref, h1_ref, c1_ref,
                h0s, c0s, h1s, c1s, *, t_last):
    t = pl.program_id(0)

    @pl.when(t == 0)
    def _init():
        z = jnp.zeros(h0s.shape, jnp.float32)
        h0s[...] = z
        c0s[...] = z
        h1s[...] = z
        c1s[...] = z

    xt = net_ref[0]
    m = len_ref[...] == t
    xt = jnp.where(m, eos_ref[...], xt)

    a0 = jnp.concatenate([xt, h0s[...]], axis=1)
    g0 = jnp.dot(a0, w0_ref[...], preferred_element_type=jnp.float32)
    g0 = g0 + b0_ref[...]
    i0 = jax.nn.sigmoid(g0[:, :_HID])
    f0 = jax.nn.sigmoid(g0[:, _HID:2 * _HID])
    u0 = jnp.tanh(g0[:, 2 * _HID:3 * _HID])
    o0 = jax.nn.sigmoid(g0[:, 3 * _HID:])
    c0n = f0 * c0s[...] + i0 * u0
    h0n = o0 * jnp.tanh(c0n)
    h0s[...] = h0n
    c0s[...] = c0n

    a1 = jnp.concatenate([h0n, h1s[...]], axis=1)
    g1 = jnp.dot(a1, w1_ref[...], preferred_element_type=jnp.float32)
    g1 = g1 + b1_ref[...]
    i1 = jax.nn.sigmoid(g1[:, :_HID])
    f1 = jax.nn.sigmoid(g1[:, _HID:2 * _HID])
    u1 = jnp.tanh(g1[:, 2 * _HID:3 * _HID])
    o1 = jax.nn.sigmoid(g1[:, 3 * _HID:])
    c1n = f1 * c1s[...] + i1 * u1
    h1n = o1 * jnp.tanh(c1n)
    h1s[...] = h1n
    c1s[...] = c1n

    ys_ref[0] = h1n

    @pl.when(t == t_last)
    def _fin():
        h0_ref[...] = h0n
        c0_ref[...] = c0n
        h1_ref[...] = h1n
        c1_ref[...] = c1n


def _lstm2(net, lengths2d, eos_row, w0, w1, b0, b1, *, interpret=False):
    tp1, b_sz, _ = net.shape
    grid = (tp1,)
    out_shape = [
        jax.ShapeDtypeStruct((tp1, b_sz, _HID), jnp.float32),
        jax.ShapeDtypeStruct((b_sz, _HID), jnp.float32),
        jax.ShapeDtypeStruct((b_sz, _HID), jnp.float32),
        jax.ShapeDtypeStruct((b_sz, _HID), jnp.float32),
        jax.ShapeDtypeStruct((b_sz, _HID), jnp.float32),
    ]
    const = lambda *_: (0, 0)
    in_specs = [
        pl.BlockSpec((1, b_sz, _EMB), lambda t: (t, 0, 0)),
        pl.BlockSpec(lengths2d.shape, const),
        pl.BlockSpec(eos_row.shape, const),
        pl.BlockSpec(w0.shape, const),
        pl.BlockSpec(w1.shape, const),
        pl.BlockSpec(b0.shape, const),
        pl.BlockSpec(b1.shape, const),
    ]
    out_specs = [
        pl.BlockSpec((1, b_sz, _HID), lambda t: (t, 0, 0)),
        pl.BlockSpec((b_sz, _HID), const),
        pl.BlockSpec((b_sz, _HID), const),
        pl.BlockSpec((b_sz, _HID), const),
        pl.BlockSpec((b_sz, _HID), const),
    ]
    scratch = [pltpu.VMEM((b_sz, _HID), jnp.float32)] * 4
    return pl.pallas_call(
        functools.partial(_lstm2_body, t_last=tp1 - 1),
        grid=grid,
        in_specs=in_specs,
        out_specs=out_specs,
        out_shape=out_shape,
        scratch_shapes=scratch,
        compiler_params=pltpu.CompilerParams(
            dimension_semantics=("arbitrary",),
        ),
        interpret=interpret,
    )(net, lengths2d, eos_row, w0, w1, b0, b1)


def kernel(inputs, lengths, emb, Wih0, Whh0, bih0, bhh0, Wih1, Whh1, bih1, bhh1):
    t_sz, b_sz = inputs.shape
    vocab = emb.shape[0] - 1

    x = jnp.concatenate(
        [inputs.astype(jnp.int32), jnp.zeros((1, b_sz), jnp.int32)], axis=0)
    idx_flat = x.reshape(-1)
    n_tok = idx_flat.shape[0]

    info = plsc.get_sparse_core_info()
    nw = info.num_cores * info.num_subcores
    n_pad = nw * _ROWS_W
    idx_pad = jnp.concatenate(
        [idx_flat, jnp.zeros((n_pad - n_tok,), jnp.int32)])
    idx3 = idx_pad.reshape(nw, _NCHUNK, _CHUNK)

    rows = _emb_gather(emb, idx3)
    net = rows[:n_tok].reshape(t_sz + 1, b_sz, _EMB)

    w0 = jnp.concatenate([Wih0, Whh0], axis=1).T
    w1 = jnp.concatenate([Wih1, Whh1], axis=1).T
    b0 = (bih0 + bhh0).reshape(1, -1)
    b1 = (bih1 + bhh1).reshape(1, -1)
    lengths2d = lengths.astype(jnp.int32).reshape(b_sz, 1)
    eos_row = emb[vocab].reshape(1, _EMB)

    ys1, h0, c0, h1, c1 = _lstm2(net, lengths2d, eos_row, w0, w1, b0, b1)
    hN = jnp.stack([h0, h1], axis=0)
    cN = jnp.stack([c0, c1], axis=0)
    return ys1, hN, cN


# speedup vs baseline: 2.8464x; 1.0311x over previous
import functools

import jax
import jax.numpy as jnp
from jax import lax
from jax.experimental import pallas as pl
from jax.experimental.pallas import tpu as pltpu
from jax.experimental.pallas import tpu_sc as plsc

_EMB = 128
_HID = 512
_CHUNK = 104
_NCHUNK = 2
_ROWS_W = _CHUNK * _NCHUNK


def _emb_gather(table, idx3):
    nw = idx3.shape[0]
    mesh = plsc.VectorSubcoreMesh(core_axis_name="c", subcore_axis_name="s")

    @functools.partial(
        pl.kernel,
        mesh=mesh,
        out_type=jax.ShapeDtypeStruct((nw * _ROWS_W, _EMB), jnp.float32),
        scratch_types=[
            pltpu.VMEM((_NCHUNK, _CHUNK), jnp.int32),
            pltpu.VMEM((_ROWS_W, _EMB), jnp.float32),
            pltpu.SemaphoreType.DMA,
        ],
    )
    def gather_kernel(table_hbm, idx_hbm, out_hbm, idx_v, rows_v, sem):
        nc = lax.axis_size("c")
        wid = lax.axis_index("s") * nc + lax.axis_index("c")
        pltpu.sync_copy(idx_hbm.at[wid], idx_v)
        cps = []
        for j in range(_NCHUNK):
            cps.append(
                pltpu.async_copy(
                    table_hbm.at[idx_v.at[j]],
                    rows_v.at[pl.ds(j * _CHUNK, _CHUNK)],
                    sem,
                )
            )
        for cp in cps:
            cp.wait()
        pltpu.sync_copy(rows_v, out_hbm.at[pl.ds(wid * _ROWS_W, _ROWS_W)])

    return gather_kernel(table, idx3)


def _lstm2_body(net_ref, len_ref, eos_ref, w0_ref, w1_ref, b0_ref, b1_ref,
                ys_ref, h0_ref, c0_ref, h1_ref, c1_ref,
                h0s, c0s, h1s, c1s, *, t_last):
    t = pl.program_id(0)

    @pl.when(t == 0)
    def _init():
        z = jnp.zeros(h0s.shape, jnp.float32)
        h0s[...] = z
        c0s[...] = z
        h1s[...] = z
        c1s[...] = z

    h0_prev = h0s[...]
    h1_prev = h1s[...]
    c0_prev = c0s[...]
    c1_prev = c1s[...]

    xt = net_ref[0]
    m = len_ref[...] == t
    xt = jnp.where(m, eos_ref[...], xt)
    a0 = jnp.concatenate([xt, h0_prev], axis=1).astype(jnp.bfloat16)
    g0 = jnp.dot(a0, w0_ref[...], preferred_element_type=jnp.float32)
    g0 = g0 + b0_ref[...]
    i0 = jax.nn.sigmoid(g0[:, :_HID])
    f0 = jax.nn.sigmoid(g0[:, _HID:2 * _HID])
    u0 = jnp.tanh(g0[:, 2 * _HID:3 * _HID])
    o0 = jax.nn.sigmoid(g0[:, 3 * _HID:])
    c0n = f0 * c0_prev + i0 * u0
    h0n = o0 * jnp.tanh(c0n)

    a1 = jnp.concatenate([h0_prev, h1_prev], axis=1).astype(jnp.bfloat16)
    g1 = jnp.dot(a1, w1_ref[...], preferred_element_type=jnp.float32)
    g1 = g1 + b1_ref[...]
    i1 = jax.nn.sigmoid(g1[:, :_HID])
    f1 = jax.nn.sigmoid(g1[:, _HID:2 * _HID])
    u1 = jnp.tanh(g1[:, 2 * _HID:3 * _HID])
    o1 = jax.nn.sigmoid(g1[:, 3 * _HID:])
    c1n = f1 * c1_prev + i1 * u1
    h1n = o1 * jnp.tanh(c1n)
    ys_ref[0] = h1n

    keep0 = t <= t_last
    keep1 = t >= 1
    h0s[...] = jnp.where(keep0, h0n, h0_prev)
    c0s[...] = jnp.where(keep0, c0n, c0_prev)
    h1s[...] = jnp.where(keep1, h1n, h1_prev)
    c1s[...] = jnp.where(keep1, c1n, c1_prev)

    @pl.when(t == t_last + 1)
    def _fin():
        h0_ref[...] = h0s[...]
        c0_ref[...] = c0s[...]
        h1_ref[...] = h1s[...]
        c1_ref[...] = c1s[...]


def _lstm2(net, lengths2d, eos_row, w0, w1, b0, b1, *, interpret=False):
    tp1, b_sz, _ = net.shape
    t_last = tp1 - 1
    grid = (tp1 + 1,)
    out_shape = [
        jax.ShapeDtypeStruct((tp1, b_sz, _HID), jnp.float32),
        jax.ShapeDtypeStruct((b_sz, _HID), jnp.float32),
        jax.ShapeDtypeStruct((b_sz, _HID), jnp.float32),
        jax.ShapeDtypeStruct((b_sz, _HID), jnp.float32),
        jax.ShapeDtypeStruct((b_sz, _HID), jnp.float32),
    ]
    const = lambda *_: (0, 0)
    in_specs = [
        pl.BlockSpec((1, b_sz, _EMB), lambda t: (jnp.minimum(t, t_last), 0, 0)),
        pl.BlockSpec(lengths2d.shape, const),
        pl.BlockSpec(eos_row.shape, const),
        pl.BlockSpec(w0.shape, const),
        pl.BlockSpec(w1.shape, const),
        pl.BlockSpec(b0.shape, const),
        pl.BlockSpec(b1.shape, const),
    ]
    out_specs = [
        pl.BlockSpec((1, b_sz, _HID), lambda t: (jnp.maximum(t - 1, 0), 0, 0)),
        pl.BlockSpec((b_sz, _HID), const),
        pl.BlockSpec((b_sz, _HID), const),
        pl.BlockSpec((b_sz, _HID), const),
        pl.BlockSpec((b_sz, _HID), const),
    ]
    scratch = [pltpu.VMEM((b_sz, _HID), jnp.float32)] * 4
    return pl.pallas_call(
        functools.partial(_lstm2_body, t_last=t_last),
        grid=grid,
        in_specs=in_specs,
        out_specs=out_specs,
        out_shape=out_shape,
        scratch_shapes=scratch,
        compiler_params=pltpu.CompilerParams(
            dimension_semantics=("arbitrary",),
        ),
        interpret=interpret,
    )(net, lengths2d, eos_row, w0, w1, b0, b1)


def kernel(inputs, lengths, emb, Wih0, Whh0, bih0, bhh0, Wih1, Whh1, bih1, bhh1):
    t_sz, b_sz = inputs.shape
    vocab = emb.shape[0] - 1

    x = jnp.concatenate(
        [inputs.astype(jnp.int32), jnp.zeros((1, b_sz), jnp.int32)], axis=0)
    idx_flat = x.reshape(-1)
    n_tok = idx_flat.shape[0]

    info = plsc.get_sparse_core_info()
    nw = info.num_cores * info.num_subcores
    n_pad = nw * _ROWS_W
    idx_pad = jnp.concatenate(
        [idx_flat, jnp.zeros((n_pad - n_tok,), jnp.int32)])
    idx3 = idx_pad.reshape(nw, _NCHUNK, _CHUNK)

    rows = _emb_gather(emb, idx3)
    net = rows[:n_tok].reshape(t_sz + 1, b_sz, _EMB)

    w0 = jnp.concatenate([Wih0, Whh0], axis=1).T.astype(jnp.bfloat16)
    w1 = jnp.concatenate([Wih1, Whh1], axis=1).T.astype(jnp.bfloat16)
    b0 = (bih0 + bhh0).reshape(1, -1)
    b1 = (bih1 + bhh1).reshape(1, -1)
    lengths2d = lengths.astype(jnp.int32).reshape(b_sz, 1)
    eos_row = emb[vocab].reshape(1, _EMB)

    ys1, h0, c0, h1, c1 = _lstm2(net, lengths2d, eos_row, w0, w1, b0, b1)
    hN = jnp.stack([h0, h1], axis=0)
    cN = jnp.stack([c0, c1], axis=0)
    return ys1, hN, cN


# speedup vs baseline: 2.8802x; 1.0119x over previous
import functools

import jax
import jax.numpy as jnp
from jax import lax
from jax.experimental import pallas as pl
from jax.experimental.pallas import tpu as pltpu
from jax.experimental.pallas import tpu_sc as plsc

_EMB = 128
_HID = 512
_CHUNK = 104
_NCHUNK = 2
_ROWS_W = _CHUNK * _NCHUNK


def _emb_gather(table, idx3):
    nw = idx3.shape[0]
    mesh = plsc.VectorSubcoreMesh(core_axis_name="c", subcore_axis_name="s")

    @functools.partial(
        pl.kernel,
        mesh=mesh,
        out_type=jax.ShapeDtypeStruct((nw * _ROWS_W, _EMB), jnp.float32),
        scratch_types=[
            pltpu.VMEM((_NCHUNK, _CHUNK), jnp.int32),
            pltpu.VMEM((_ROWS_W, _EMB), jnp.float32),
            pltpu.SemaphoreType.DMA,
        ],
    )
    def gather_kernel(table_hbm, idx_hbm, out_hbm, idx_v, rows_v, sem):
        nc = lax.axis_size("c")
        wid = lax.axis_index("s") * nc + lax.axis_index("c")
        pltpu.sync_copy(idx_hbm.at[wid], idx_v)
        cps = []
        for j in range(_NCHUNK):
            cps.append(
                pltpu.async_copy(
                    table_hbm.at[idx_v.at[j]],
                    rows_v.at[pl.ds(j * _CHUNK, _CHUNK)],
                    sem,
                )
            )
        for cp in cps:
            cp.wait()
        pltpu.sync_copy(rows_v, out_hbm.at[pl.ds(wid * _ROWS_W, _ROWS_W)])

    return gather_kernel(table, idx3)


def _cell(a_bf16, c_prev, w_ref, b_ref):
    g = jnp.dot(a_bf16, w_ref[...], preferred_element_type=jnp.float32)
    g = g + b_ref[...]
    i = jax.nn.sigmoid(g[:, :_HID])
    f = jax.nn.sigmoid(g[:, _HID:2 * _HID])
    u = jnp.tanh(g[:, 2 * _HID:3 * _HID])
    o = jax.nn.sigmoid(g[:, 3 * _HID:])
    c = f * c_prev + i * u
    h = o * jnp.tanh(c)
    return h, c


def _lstm2_body(net_ref, len_ref, eos_ref, w0_ref, w1_ref, b0_ref, b1_ref,
                ys_ref, h0_ref, c0_ref, h1_ref, c1_ref, *, t_last):
    b_sz = h0_ref.shape[0]
    z = jnp.zeros((b_sz, _HID), jnp.float32)

    def step(t, carry):
        h0_prev, c0_prev, h1_prev, c1_prev = carry

        xt = net_ref[pl.ds(jnp.minimum(t, t_last), 1)][0]
        xt = jnp.where(len_ref[...] == t, eos_ref[...], xt)
        a0 = jnp.concatenate([xt, h0_prev], axis=1).astype(jnp.bfloat16)
        h0n, c0n = _cell(a0, c0_prev, w0_ref, b0_ref)

        a1 = jnp.concatenate([h0_prev, h1_prev],
                             axis=1).astype(jnp.bfloat16)
        h1n, c1n = _cell(a1, c1_prev, w1_ref, b1_ref)
        ys_ref[pl.ds(jnp.maximum(t - 1, 0), 1)] = h1n[None]

        keep0 = t <= t_last
        keep1 = t >= 1
        return (jnp.where(keep0, h0n, h0_prev),
                jnp.where(keep0, c0n, c0_prev),
                jnp.where(keep1, h1n, h1_prev),
                jnp.where(keep1, c1n, c1_prev))

    h0, c0, h1, c1 = lax.fori_loop(0, t_last + 2, step, (z, z, z, z))
    h0_ref[...] = h0
    c0_ref[...] = c0
    h1_ref[...] = h1
    c1_ref[...] = c1


def _lstm2(net, lengths2d, eos_row, w0, w1, b0, b1, *, interpret=False):
    tp1, b_sz, _ = net.shape
    t_last = tp1 - 1
    out_shape = [
        jax.ShapeDtypeStruct((tp1, b_sz, _HID), jnp.float32),
        jax.ShapeDtypeStruct((b_sz, _HID), jnp.float32),
        jax.ShapeDtypeStruct((b_sz, _HID), jnp.float32),
        jax.ShapeDtypeStruct((b_sz, _HID), jnp.float32),
        jax.ShapeDtypeStruct((b_sz, _HID), jnp.float32),
    ]
    return pl.pallas_call(
        functools.partial(_lstm2_body, t_last=t_last),
        out_shape=out_shape,
        interpret=interpret,
    )(net, lengths2d, eos_row, w0, w1, b0, b1)


def kernel(inputs, lengths, emb, Wih0, Whh0, bih0, bhh0, Wih1, Whh1, bih1, bhh1):
    t_sz, b_sz = inputs.shape
    vocab = emb.shape[0] - 1

    x = jnp.concatenate(
        [inputs.astype(jnp.int32), jnp.zeros((1, b_sz), jnp.int32)], axis=0)
    idx_flat = x.reshape(-1)
    n_tok = idx_flat.shape[0]

    info = plsc.get_sparse_core_info()
    nw = info.num_cores * info.num_subcores
    n_pad = nw * _ROWS_W
    idx_pad = jnp.concatenate(
        [idx_flat, jnp.zeros((n_pad - n_tok,), jnp.int32)])
    idx3 = idx_pad.reshape(nw, _NCHUNK, _CHUNK)

    rows = _emb_gather(emb, idx3)
    net = rows[:n_tok].reshape(t_sz + 1, b_sz, _EMB)

    w0 = jnp.concatenate([Wih0, Whh0], axis=1).T.astype(jnp.bfloat16)
    w1 = jnp.concatenate([Wih1, Whh1], axis=1).T.astype(jnp.bfloat16)
    b0 = (bih0 + bhh0).reshape(1, -1)
    b1 = (bih1 + bhh1).reshape(1, -1)
    lengths2d = lengths.astype(jnp.int32).reshape(b_sz, 1)
    eos_row = emb[vocab].reshape(1, _EMB)

    ys1, h0, c0, h1, c1 = _lstm2(net, lengths2d, eos_row, w0, w1, b0, b1)
    hN = jnp.stack([h0, h1], axis=0)
    cN = jnp.stack([c0, c1], axis=0)
    return ys1, hN, cN
